# baseline (speedup 1.0000x reference)
import functools

import jax
import jax.numpy as jnp
import numpy as np
from jax import lax
from jax.experimental import pallas as pl
from jax.experimental.pallas import tpu as pltpu
from jax.experimental.pallas import tpu_sc as plsc

B, F, D = 4096, 100, 128
R = B * F
NC, NS = 2, 16
NW = NC * NS
ROWS_PER_W = R // NW
CHUNK = 128
STEPS = ROWS_PER_W // CHUNK

_g = np.arange(R, dtype=np.int64)
_IDX = ((_g // F) * F + (F - 1 - (_g % F))).astype(np.int32)

_mesh = plsc.VectorSubcoreMesh(core_axis_name="c", subcore_axis_name="s")


@functools.partial(
    pl.kernel,
    mesh=_mesh,
    out_type=jax.ShapeDtypeStruct((R, D), jnp.float32),
    scratch_types=[
        pltpu.VMEM((CHUNK,), jnp.int32),
        pltpu.VMEM((CHUNK, D), jnp.float32),
        pltpu.SemaphoreType.DMA,
    ],
)
def _reorder(x_hbm, idx_hbm, out_hbm, idx_v, buf, sem):
    wid = lax.axis_index("s") * NC + lax.axis_index("c")
    base = wid * ROWS_PER_W

    def body(s, carry):
        row0 = base + s * CHUNK
        pltpu.sync_copy(idx_hbm.at[pl.ds(row0, CHUNK)], idx_v)
        pltpu.async_copy(x_hbm.at[idx_v], buf, sem).wait()
        pltpu.sync_copy(buf, out_hbm.at[pl.ds(row0, CHUNK)])
        return carry

    lax.fori_loop(0, STEPS, body, 0)


def kernel(x):
    out = _reorder(x.reshape(R, D), _IDX)
    return out.reshape(B, F, D)


# speedup vs baseline: 1.1303x; 1.1303x over previous
import functools

import jax
import jax.numpy as jnp
import numpy as np
from jax import lax
from jax.experimental import pallas as pl
from jax.experimental.pallas import tpu as pltpu
from jax.experimental.pallas import tpu_sc as plsc

B, F, D = 4096, 100, 128
R = B * F
NC, NS = 2, 16
NW = NC * NS
ROWS_PER_W = R // NW
CHUNK = 128
STEPS = ROWS_PER_W // CHUNK

_g = np.arange(R, dtype=np.int64)
_IDX = ((_g // F) * F + (F - 1 - (_g % F))).astype(np.int32)

_mesh = plsc.VectorSubcoreMesh(core_axis_name="c", subcore_axis_name="s")

NBUF = 4
G = STEPS // NBUF


@functools.partial(
    pl.kernel,
    mesh=_mesh,
    out_type=jax.ShapeDtypeStruct((R, D), jnp.float32),
    scratch_types=(
        [pltpu.VMEM((CHUNK,), jnp.int32) for _ in range(NBUF)]
        + [pltpu.VMEM((CHUNK, D), jnp.float32) for _ in range(NBUF)]
        + [pltpu.SemaphoreType.DMA for _ in range(3 * NBUF)]
    ),
)
def _reorder(x_hbm, idx_hbm, out_hbm, *refs):
    idx_v = refs[0:NBUF]
    buf = refs[NBUF:2 * NBUF]
    isem = refs[2 * NBUF:3 * NBUF]
    gsem = refs[3 * NBUF:4 * NBUF]
    wsem = refs[4 * NBUF:5 * NBUF]
    wid = lax.axis_index("s") * NC + lax.axis_index("c")
    base = wid * ROWS_PER_W

    def start_idx(b, s):
        pltpu.async_copy(idx_hbm.at[pl.ds(base + s * CHUNK, CHUNK)],
                         idx_v[b], isem[b])

    def wait_idx(b):
        pltpu.make_async_copy(idx_hbm.at[pl.ds(base, CHUNK)],
                              idx_v[b], isem[b]).wait()

    def start_gather(b):
        pltpu.async_copy(x_hbm.at[idx_v[b]], buf[b], gsem[b])

    def wait_gather(b):
        pltpu.make_async_copy(x_hbm.at[idx_v[b]], buf[b], gsem[b]).wait()

    def start_write(b, s):
        pltpu.async_copy(buf[b], out_hbm.at[pl.ds(base + s * CHUNK, CHUNK)],
                         wsem[b])

    def wait_write(b):
        pltpu.make_async_copy(buf[b], out_hbm.at[pl.ds(base, CHUNK)],
                              wsem[b]).wait()

    for b in range(NBUF):
        start_idx(b, b)
    for b in range(NBUF):
        wait_idx(b)
        start_gather(b)

    def body(g, carry):
        s0 = g * NBUF
        for b in range(NBUF):
            wait_gather(b)
            start_write(b, s0 + b)

        @pl.when(g < G - 1)
        def _next():
            for b in range(NBUF):
                start_idx(b, s0 + NBUF + b)
            for b in range(NBUF):
                wait_write(b)
                wait_idx(b)
                start_gather(b)

        return carry

    lax.fori_loop(0, G, body, 0)
    for b in range(NBUF):
        wait_write(b)


def kernel(x):
    out = _reorder(x.reshape(R, D), _IDX)
    return out.reshape(B, F, D)


# speedup vs baseline: 2.0652x; 1.8271x over previous
import functools

import jax
import jax.numpy as jnp
import numpy as np
from jax import lax
from jax.experimental import pallas as pl
from jax.experimental.pallas import tpu as pltpu
from jax.experimental.pallas import tpu_sc as plsc

B, F, D = 4096, 100, 128
NC, NS = 2, 16
NW = NC * NS
BATCHES_PER_W = B // NW
NBUF = 4
G = BATCHES_PER_W // NBUF

_IDX = np.arange(F - 1, -1, -1, dtype=np.int32)

_mesh = plsc.VectorSubcoreMesh(core_axis_name="c", subcore_axis_name="s")


@functools.partial(
    pl.kernel,
    mesh=_mesh,
    out_type=jax.ShapeDtypeStruct((B, F, D), jnp.float32),
    scratch_types=(
        [pltpu.VMEM((F,), jnp.int32)]
        + [pltpu.VMEM((F, D), jnp.float32) for _ in range(NBUF)]
        + [pltpu.SemaphoreType.DMA for _ in range(2 * NBUF)]
    ),
)
def _reorder(x_hbm, idx_hbm, out_hbm, idx_v, *refs):
    buf = refs[0:NBUF]
    gsem = refs[NBUF:2 * NBUF]
    wsem = refs[2 * NBUF:3 * NBUF]
    wid = lax.axis_index("s") * NC + lax.axis_index("c")
    base = wid * BATCHES_PER_W

    pltpu.sync_copy(idx_hbm, idx_v)

    def start_gather(k, b):
        pltpu.async_copy(x_hbm.at[base + b].at[idx_v], buf[k], gsem[k])

    def wait_gather(k):
        pltpu.make_async_copy(x_hbm.at[base].at[idx_v], buf[k], gsem[k]).wait()

    def start_write(k, b):
        pltpu.async_copy(buf[k], out_hbm.at[base + b], wsem[k])

    def wait_write(k):
        pltpu.make_async_copy(buf[k], out_hbm.at[base], wsem[k]).wait()

    for k in range(NBUF):
        start_gather(k, k)

    def body(g, carry):
        b0 = g * NBUF
        for k in range(NBUF):
            wait_gather(k)
            start_write(k, b0 + k)

        @pl.when(g < G - 1)
        def _next():
            for k in range(NBUF):
                wait_write(k)
                start_gather(k, b0 + NBUF + k)

        return carry

    lax.fori_loop(0, G, body, 0)
    for k in range(NBUF):
        wait_write(k)


def kernel(x):
    return _reorder(x, _IDX)


# speedup vs baseline: 2.0707x; 1.0027x over previous
import functools

import jax
import jax.numpy as jnp
import numpy as np
from jax import lax
from jax.experimental import pallas as pl
from jax.experimental.pallas import tpu as pltpu
from jax.experimental.pallas import tpu_sc as plsc

B, F, D = 4096, 100, 128
NC, NS = 2, 16
NW = NC * NS
BATCHES_PER_W = B // NW
NBUF = 4
G = BATCHES_PER_W // NBUF

_IDX = np.arange(F - 1, -1, -1, dtype=np.int32)

_mesh = plsc.VectorSubcoreMesh(core_axis_name="c", subcore_axis_name="s")


@functools.partial(
    pl.kernel,
    mesh=_mesh,
    out_type=jax.ShapeDtypeStruct((B, F, D), jnp.float32),
    scratch_types=(
        [pltpu.VMEM((F,), jnp.int32)]
        + [pltpu.VMEM((F, D), jnp.float32) for _ in range(NBUF)]
        + [pltpu.SemaphoreType.DMA for _ in range(2 * NBUF)]
    ),
    compiler_params=pltpu.CompilerParams(use_tc_tiling_on_sc=True),
)
def _reorder(x_hbm, idx_hbm, out_hbm, idx_v, *refs):
    buf = refs[0:NBUF]
    gsem = refs[NBUF:2 * NBUF]
    wsem = refs[2 * NBUF:3 * NBUF]
    wid = lax.axis_index("s") * NC + lax.axis_index("c")
    base = wid * BATCHES_PER_W

    pltpu.sync_copy(idx_hbm, idx_v)

    def start_gather(k, b):
        pltpu.async_copy(x_hbm.at[base + b].at[idx_v], buf[k], gsem[k])

    def wait_gather(k):
        pltpu.make_async_copy(x_hbm.at[base].at[idx_v], buf[k], gsem[k]).wait()

    def start_write(k, b):
        pltpu.async_copy(buf[k], out_hbm.at[base + b], wsem[k])

    def wait_write(k):
        pltpu.make_async_copy(buf[k], out_hbm.at[base], wsem[k]).wait()

    for k in range(NBUF):
        start_gather(k, k)

    def body(g, carry):
        b0 = g * NBUF
        for k in range(NBUF):
            wait_gather(k)
            start_write(k, b0 + k)

        @pl.when(g < G - 1)
        def _next():
            for k in range(NBUF):
                wait_write(k)
                start_gather(k, b0 + NBUF + k)

        return carry

    lax.fori_loop(0, G, body, 0)
    for k in range(NBUF):
        wait_write(k)


def kernel(x):
    return _reorder(x, _IDX)
